# Initial kernel scaffold; baseline (speedup 1.0000x reference)
import jax
import jax.numpy as jnp
from jax.experimental import pallas as pl


def kernel(x, edge_index, edge_attr, batch, solvent_fingerprint, W1, b1, W2, b2, W3, b3, Wg1, bg1, Wg2, bg2, Ws1, bs1, Ws2, bs2, Ws3, bs3, Wsf1, bsf1, Wsf2, bsf2, Wf1, bf1, Wf2, bf2, Wo, bo):
    raise NotImplementedError("write your pallas kernel here")


# speedup vs baseline: 11.9742x; 11.9742x over previous
import functools

import jax
import jax.numpy as jnp
from jax import lax
from jax.experimental import pallas as pl
from jax.experimental.pallas import tpu as pltpu
from jax.experimental.pallas import tpu_sc as plsc

N = 10000
E = 320000
B = 64
N_PAD = 10240
DUMMY = 10000
NT = 32
EPT = E // NT
NB_E = 79
EPT_P = NB_E * 128
SEG_PAD = 80
CNT_PAD = 128
F32 = jnp.float32
I32 = jnp.int32

_mesh = functools.partial(
    plsc.VectorSubcoreMesh, core_axis_name="c", subcore_axis_name="s",
    num_cores=2, num_subcores=16)


@functools.cache
def _deg_kernel():
    @functools.partial(
        pl.kernel,
        out_type=(jax.ShapeDtypeStruct((2 * N_PAD,), F32),
                  jax.ShapeDtypeStruct((2 * CNT_PAD,), F32)),
        mesh=_mesh(),
        scratch_types=dict(
            dstv=pltpu.VMEM((NB_E, 128), I32),
            bidx=pltpu.VMEM((3, 128), I32),
            ones=pltpu.VMEM((128,), F32),
            dbuf=pltpu.VMEM((640,), F32),
            degacc=pltpu.VMEM_SHARED((N_PAD,), F32),
            cacc=pltpu.VMEM_SHARED((CNT_PAD,), F32),
        ),
    )
    def deg_k(dst_hbm, bcnt_hbm, zeros1d, degf, countsf,
              dstv, bidx, ones, dbuf, degacc, cacc):
        c = lax.axis_index("c")
        s = lax.axis_index("s")
        w = c * 16 + s
        one16 = jnp.ones((16,), F32)

        def fill_ones(i, _):
            ones[pl.ds(i * 16, 16)] = one16
            return 0
        lax.fori_loop(0, 8, fill_ones, 0)

        pltpu.sync_copy(zeros1d, degacc.at[pl.ds(s * 640, 640)])

        @pl.when(s == 0)
        def _():
            pltpu.sync_copy(zeros1d.at[pl.ds(0, CNT_PAD)], cacc)

        pltpu.sync_copy(dst_hbm.at[w], dstv)
        pltpu.sync_copy(bcnt_hbm.at[w], bidx)
        plsc.subcore_barrier()

        def edge_step(j, _):
            pltpu.sync_copy(ones, degacc.at[dstv.at[j]], add=True)
            return 0
        lax.fori_loop(0, NB_E, edge_step, 0)

        def cnt_step(j, _):
            pltpu.sync_copy(ones, cacc.at[bidx.at[j]], add=True)
            return 0
        lax.fori_loop(0, 3, cnt_step, 0)

        plsc.subcore_barrier()
        pltpu.sync_copy(degacc.at[pl.ds(s * 640, 640)], dbuf)
        pltpu.sync_copy(dbuf, degf.at[pl.ds(c * N_PAD + s * 640, 640)])

        @pl.when(s == 0)
        def _():
            pltpu.sync_copy(cacc, dbuf.at[pl.ds(0, CNT_PAD)])
            pltpu.sync_copy(dbuf.at[pl.ds(0, CNT_PAD)],
                            countsf.at[pl.ds(c * CNT_PAD, CNT_PAD)])

    return deg_k


@functools.cache
def _agg_kernel():
    @functools.partial(
        pl.kernel,
        out_type=jax.ShapeDtypeStruct((2 * N_PAD, 128), F32),
        mesh=_mesh(),
        scratch_types=dict(
            srcv=pltpu.VMEM((NB_E, 128), I32),
            dstv=pltpu.VMEM((NB_E, 128), I32),
            gbuf=pltpu.VMEM((128, 128), F32),
            acc=pltpu.VMEM_SHARED((N_PAD, 128), F32),
        ),
    )
    def agg_k(table, src_hbm, dst_hbm, zeros2d, out, srcv, dstv, gbuf, acc):
        c = lax.axis_index("c")
        s = lax.axis_index("s")
        pltpu.sync_copy(zeros2d, acc.at[pl.ds(s * 640, 640)])
        plsc.subcore_barrier()
        for ep in range(2):
            w = ep * 16 + s
            pltpu.sync_copy(dst_hbm.at[w], dstv)
            pltpu.sync_copy(src_hbm.at[c * NT + w], srcv)

            def edge_step(j, _):
                pltpu.sync_copy(table.at[srcv.at[j]], gbuf)
                pltpu.sync_copy(gbuf, acc.at[dstv.at[j]], add=True)
                return 0
            lax.fori_loop(0, NB_E, edge_step, 0)
        plsc.subcore_barrier()
        for q in range(5):
            pltpu.sync_copy(acc.at[pl.ds(s * 640 + q * 128, 128)], gbuf)
            pltpu.sync_copy(
                gbuf, out.at[pl.ds(c * N_PAD + s * 640 + q * 128, 128)])

    return agg_k


@functools.cache
def _pool_kernel():
    @functools.partial(
        pl.kernel,
        out_type=(jax.ShapeDtypeStruct((NT * SEG_PAD * 512,), F32),
                  jax.ShapeDtypeStruct((NT * SEG_PAD * 512,), F32)),
        mesh=_mesh(),
        compiler_params=pltpu.CompilerParams(needs_layout_passes=False),
        scratch_types=dict(
            bvm=pltpu.VMEM((640,), I32),
            rowbuf=pltpu.VMEM((32, 512), F32),
            sumacc=pltpu.VMEM((SEG_PAD * 512,), F32),
            maxacc=pltpu.VMEM((SEG_PAD * 512,), F32),
        ),
    )
    def pool_k(h3, s3, bp1d, sumpf, maxpf, bvm, rowbuf, sumacc, maxacc):
        c = lax.axis_index("c")
        s = lax.axis_index("s")
        pltpu.sync_copy(bp1d.at[s], bvm)
        z16 = jnp.zeros((16,), F32)

        def zrow(r, _):
            sumacc[pl.ds(r * 16, 16)] = z16
            maxacc[pl.ds(r * 16, 16)] = z16
            return 0
        lax.fori_loop(0, SEG_PAD * 32, zrow, 0)

        lane = lax.iota(I32, 16)

        def branch(tbl):
            def qstep(q, _q):
                pltpu.sync_copy(tbl.at[pl.ds(s * 640 + q * 32, 32)],
                                rowbuf)

                def grp(t, _):
                    segv = bvm[pl.ds(q * 32 + t * 16, 16)]

                    def row(j, _2):
                        r = t * 16 + j
                        splat = segv[jnp.full((16,), 0, I32) + j]
                        base = splat * 512 + lane
                        for k in range(32):
                            idxv = base + (k * 16)
                            val = rowbuf[r, pl.ds(k * 16, 16)]
                            cs = plsc.load_gather(sumacc, [idxv])
                            plsc.store_scatter(sumacc, [idxv], cs + val)
                            cm = plsc.load_gather(maxacc, [idxv])
                            plsc.store_scatter(maxacc, [idxv],
                                               jnp.maximum(cm, val))
                        return 0
                    lax.fori_loop(0, 16, row, 0)
                    return 0
                lax.fori_loop(0, 2, grp, 0)
                return 0
            lax.fori_loop(0, 20, qstep, 0)

        @pl.when(c == 0)
        def _():
            branch(h3)

        @pl.when(c == 1)
        def _():
            branch(s3)

        off = (c * 16 + s) * (SEG_PAD * 512)
        pltpu.sync_copy(sumacc, sumpf.at[pl.ds(off, SEG_PAD * 512)])
        pltpu.sync_copy(maxacc, maxpf.at[pl.ds(off, SEG_PAD * 512)])

    return pool_k


def _prep_body(x_ref, sf_ref, d0_ref, d1_ref, xp_ref, dinv_ref):
    deg = d0_ref[...] + d1_ref[...] + 1.0
    dv = lax.rsqrt(deg)
    dinv_ref[...] = dv
    xp_ref[0] = dv * x_ref[...]
    xp_ref[1] = dv * sf_ref[...]


@functools.cache
def _prep_call():
    bn = 640
    grid = N_PAD // bn
    return pl.pallas_call(
        _prep_body,
        grid=(grid,),
        in_specs=[
            pl.BlockSpec((bn, 128), lambda i: (i, 0)),
            pl.BlockSpec((bn, 128), lambda i: (i, 0)),
            pl.BlockSpec((bn, 1), lambda i: (i, 0)),
            pl.BlockSpec((bn, 1), lambda i: (i, 0)),
        ],
        out_specs=[
            pl.BlockSpec((2, bn, 128), lambda i: (0, i, 0)),
            pl.BlockSpec((bn, 1), lambda i: (i, 0)),
        ],
        out_shape=[
            jax.ShapeDtypeStruct((2, N_PAD, 128), F32),
            jax.ShapeDtypeStruct((N_PAD, 1), F32),
        ],
    )


@functools.cache
def _layer_call(cb, f_out, scale_out):
    bn = 640
    grid = N_PAD // bn
    ko = f_out // 128

    def body(*refs):
        g_refs = refs[:cb]
        xp_ref, dinv_ref, wh_ref, ws_ref, bh_ref, bs_ref = refs[cb:cb + 6]
        out_refs = refs[cb + 6:]
        dv = dinv_ref[...]
        gh = jnp.concatenate([g_refs[k][0] for k in range(cb)], axis=1)
        gs = jnp.concatenate([g_refs[k][1] for k in range(cb)], axis=1)
        xh = jnp.concatenate([xp_ref[2 * k] for k in range(cb)], axis=1)
        xs = jnp.concatenate([xp_ref[2 * k + 1] for k in range(cb)], axis=1)
        uh = dv * (gh + xh)
        us = dv * (gs + xs)
        h = jax.nn.relu(jnp.dot(uh, wh_ref[...],
                                preferred_element_type=F32) + bh_ref[...])
        sb = jax.nn.relu(jnp.dot(us, ws_ref[...],
                                 preferred_element_type=F32) + bs_ref[...])
        if scale_out:
            h = dv * h
            sb = dv * sb
            o = out_refs[0]
            for k in range(ko):
                o[2 * k] = h[:, k * 128:(k + 1) * 128]
                o[2 * k + 1] = sb[:, k * 128:(k + 1) * 128]
        else:
            out_refs[0][...] = h
            out_refs[1][...] = sb

    if scale_out:
        out_specs = [pl.BlockSpec((2 * ko, bn, 128), lambda i: (0, i, 0))]
        out_shape = [jax.ShapeDtypeStruct((2 * ko, N_PAD, 128), F32)]
    else:
        out_specs = [pl.BlockSpec((bn, f_out), lambda i: (i, 0))] * 2
        out_shape = [jax.ShapeDtypeStruct((N_PAD, f_out), F32)] * 2

    return pl.pallas_call(
        body,
        grid=(grid,),
        in_specs=[pl.BlockSpec((2, bn, 128), lambda i: (0, i, 0))] * cb + [
            pl.BlockSpec((2 * cb, bn, 128), lambda i: (0, i, 0)),
            pl.BlockSpec((bn, 1), lambda i: (i, 0)),
            pl.BlockSpec((cb * 128, f_out), lambda i: (0, 0)),
            pl.BlockSpec((cb * 128, f_out), lambda i: (0, 0)),
            pl.BlockSpec((1, f_out), lambda i: (0, 0)),
            pl.BlockSpec((1, f_out), lambda i: (0, 0)),
        ],
        out_specs=out_specs,
        out_shape=out_shape,
    )


def _head_body(sump_h, sump_s, maxp_h, maxp_s, c0, c1,
               wg1, bg1, wg2, bg2, ws1, bs1, ws2, bs2,
               wf1, bf1, wf2, bf2, wo, bo, out):
    cnt = jnp.maximum(c0[:B] + c1[:B], 1.0)
    sum_h = jnp.sum(sump_h[...], axis=0)
    sum_s = jnp.sum(sump_s[...], axis=0)
    max_h = jnp.max(maxp_h[...], axis=0)
    max_s = jnp.max(maxp_s[...], axis=0)
    gin = jnp.concatenate([sum_h[:B] / cnt, max_h[:B]], axis=1)
    sin = jnp.concatenate([sum_s[:B] / cnt, max_s[:B]], axis=1)

    hi = lax.Precision.HIGHEST

    def mlp(z, w1, b1, w2, b2):
        z = jax.nn.relu(jnp.dot(z, w1[...], precision=hi,
                                preferred_element_type=F32) + b1[...])
        return jax.nn.relu(jnp.dot(z, w2[...], precision=hi,
                                   preferred_element_type=F32) + b2[...])

    g = mlp(gin, wg1, bg1, wg2, bg2)
    s = mlp(sin, ws1, bs1, ws2, bs2)
    z = jnp.concatenate([g, s], axis=1)
    z = mlp(z, wf1, bf1, wf2, bf2)
    out[...] = jnp.dot(z, wo[...], precision=hi,
                       preferred_element_type=F32) + bo[...]


@functools.cache
def _head_call():
    return pl.pallas_call(
        _head_body,
        out_shape=jax.ShapeDtypeStruct((B, 1), F32),
    )


def kernel(x, edge_index, edge_attr, batch, solvent_fingerprint,
           W1, b1, W2, b2, W3, b3, Wg1, bg1, Wg2, bg2,
           Ws1, bs1, Ws2, bs2, Ws3, bs3, Wsf1, bsf1, Wsf2, bsf2,
           Wf1, bf1, Wf2, bf2, Wo, bo):
    src = edge_index[0].astype(I32)
    dst = edge_index[1].astype(I32)
    pad_e = EPT_P - EPT
    src_t = jnp.pad(src.reshape(NT, EPT), ((0, 0), (0, pad_e)))
    dst_t = jnp.pad(dst.reshape(NT, EPT), ((0, 0), (0, pad_e)),
                    constant_values=DUMMY).reshape(NT, NB_E, 128)
    src2 = jnp.stack([src_t, src_t + N_PAD])
    src12 = src2.reshape(2 * NT, NB_E, 128)
    src3b = (src2 + 2 * N_PAD).reshape(2 * NT, NB_E, 128)

    bp = jnp.concatenate(
        [batch.astype(I32), jnp.full((N_PAD - N,), B, I32)])
    bp1d = bp.reshape(16, 640)
    bcnt = jnp.concatenate(
        [bp.reshape(NT, 320), jnp.full((NT, 64), B, I32)],
        axis=1).reshape(NT, 3, 128)

    zeros1d = jnp.zeros((640,), F32)
    zeros2d = jnp.zeros((640, 128), F32)

    degf, countsf = _deg_kernel()(dst_t, bcnt, zeros1d)
    deg0 = degf[:N_PAD].reshape(N_PAD, 1)
    deg1 = degf[N_PAD:].reshape(N_PAD, 1)
    c0 = countsf[:SEG_PAD].reshape(SEG_PAD, 1)
    c1 = countsf[CNT_PAD:CNT_PAD + SEG_PAD].reshape(SEG_PAD, 1)

    x_p = jnp.pad(x, ((0, N_PAD - N), (0, 0)))
    sf_p = jnp.pad(solvent_fingerprint, ((0, N_PAD - N), (0, 0)))
    xp12, dinv = _prep_call()(x_p, sf_p, deg0, deg1)
    xp_flat = xp12.reshape(2 * N_PAD, 128)

    agg = _agg_kernel()
    g1 = agg(xp_flat, src12, dst_t, zeros2d).reshape(2, N_PAD, 128)
    (xp2,) = _layer_call(1, 128, True)(
        g1, xp12, dinv, W1, Ws1, b1.reshape(1, -1), bs1.reshape(1, -1))
    g2 = agg(xp2.reshape(2 * N_PAD, 128), src12, dst_t,
             zeros2d).reshape(2, N_PAD, 128)
    (xp3,) = _layer_call(1, 256, True)(
        g2, xp2, dinv, W2, Ws2, b2.reshape(1, -1), bs2.reshape(1, -1))
    xp3_flat = xp3.reshape(4 * N_PAD, 128)
    g3a = agg(xp3_flat, src12, dst_t, zeros2d).reshape(2, N_PAD, 128)
    g3b = agg(xp3_flat, src3b, dst_t, zeros2d).reshape(2, N_PAD, 128)
    h3, s3 = _layer_call(2, 512, False)(
        g3a, g3b, xp3, dinv, W3, Ws3, b3.reshape(1, -1), bs3.reshape(1, -1))

    sumpf, maxpf = _pool_kernel()(h3, s3, bp1d)
    half = 16 * SEG_PAD * 512
    sump_h = sumpf[:half].reshape(16, SEG_PAD, 512)
    sump_s = sumpf[half:].reshape(16, SEG_PAD, 512)
    maxp_h = maxpf[:half].reshape(16, SEG_PAD, 512)
    maxp_s = maxpf[half:].reshape(16, SEG_PAD, 512)
    out = _head_call()(
        sump_h, sump_s, maxp_h, maxp_s,
        c0, c1,
        Wg1, bg1.reshape(1, -1), Wg2, bg2.reshape(1, -1),
        Wsf1, bsf1.reshape(1, -1), Wsf2, bsf2.reshape(1, -1),
        Wf1, bf1.reshape(1, -1), Wf2, bf2.reshape(1, -1),
        Wo, bo.reshape(1, -1))
    return out
